# baseline (speedup 1.0000x reference)
import functools

import jax
import jax.numpy as jnp
from jax import lax
from jax.experimental import pallas as pl
from jax.experimental.pallas import tpu as pltpu
from jax.experimental.pallas import tpu_sc as plsc

_NUM_EMB = 1024
_EMB_DIM = 64
_COMMIT = 0.25
_TBLK = 2048


def _vq_body(flat_ref, w_ref, w2_ref, wsq_ref, iota_ref, enc_ref, qst_ref,
             idx_ref, loss_ref, sse_ref):
    i = pl.program_id(0)
    nsteps = pl.num_programs(0)
    xb = flat_ref[...]
    w = w_ref[...]
    xsq = jnp.sum(xb * xb, axis=1, keepdims=True)
    m2 = lax.dot_general(xb, w2_ref[...], (((1,), (1,)), ((), ())),
                         preferred_element_type=jnp.float32)
    dist = (xsq + wsq_ref[...]) - m2
    dmin = jnp.min(dist, axis=1, keepdims=True)
    iota = iota_ref[...]
    idx = jnp.min(jnp.where(dist == dmin, iota, float(_NUM_EMB)),
                  axis=1, keepdims=True)
    enc = (iota == idx).astype(jnp.float32)
    enc_ref[...] = enc
    idx_ref[...] = idx.astype(jnp.int32)
    q = lax.dot_general(enc, w, (((1,), (0,)), ((), ())),
                        preferred_element_type=jnp.float32)
    d = q - xb
    qst_ref[...] = xb + d
    sse_part = jnp.sum(d * d)

    @pl.when(i == 0)
    def _init():
        sse_ref[0] = sse_part

    @pl.when(i != 0)
    def _acc():
        sse_ref[0] += sse_part

    @pl.when(i == nsteps - 1)
    def _fin():
        n_tok = nsteps * _TBLK
        mean = sse_ref[0] / (n_tok * _EMB_DIM)
        loss_ref[...] = jnp.reshape(mean + _COMMIT * mean, (1, 1))


_SC_MESH = plsc.VectorSubcoreMesh(core_axis_name="c", subcore_axis_name="s")
_PER_WORKER = 1024


@functools.partial(
    pl.kernel,
    mesh=_SC_MESH,
    out_type=jax.ShapeDtypeStruct((2, _NUM_EMB), jnp.float32),
    scratch_types=[
        pltpu.VMEM((_PER_WORKER,), jnp.int32),
        pltpu.VMEM((_PER_WORKER,), jnp.float32),
        pltpu.VMEM_SHARED((_NUM_EMB,), jnp.float32),
    ],
)
def _hist_sc(idx_hbm, ones_hbm, zeros_hbm, out_hbm, idx_v, ones_v, shared):
    c = lax.axis_index("c")
    s = lax.axis_index("s")
    base = (c * 16 + s) * _PER_WORKER
    pltpu.sync_copy(idx_hbm.at[pl.ds(base, _PER_WORKER)], idx_v)
    pltpu.sync_copy(ones_hbm, ones_v)

    @pl.when(s == 0)
    def _zero():
        pltpu.sync_copy(zeros_hbm, shared)

    plsc.subcore_barrier()
    pltpu.sync_copy(ones_v, shared.at[idx_v], add=True)
    plsc.subcore_barrier()

    @pl.when(s == 0)
    def _drain():
        pltpu.sync_copy(shared, out_hbm.at[c])


def kernel(inputs, W):
    B, C, H, Wd = inputs.shape
    x = jnp.transpose(inputs, (0, 2, 3, 1))
    flat = x.reshape(-1, C)
    N = flat.shape[0]
    wsq = jnp.sum(W ** 2, axis=1).reshape(1, _NUM_EMB)
    w2 = W + W
    iota = lax.broadcasted_iota(jnp.float32, (1, _NUM_EMB), 1)
    grid = N // _TBLK

    enc, qst, idx, loss = pl.pallas_call(
        _vq_body,
        grid=(grid,),
        in_specs=[
            pl.BlockSpec((_TBLK, C), lambda i: (i, 0)),
            pl.BlockSpec((_NUM_EMB, C), lambda i: (0, 0)),
            pl.BlockSpec((_NUM_EMB, C), lambda i: (0, 0)),
            pl.BlockSpec((1, _NUM_EMB), lambda i: (0, 0)),
            pl.BlockSpec((1, _NUM_EMB), lambda i: (0, 0)),
        ],
        out_specs=[
            pl.BlockSpec((_TBLK, _NUM_EMB), lambda i: (i, 0)),
            pl.BlockSpec((_TBLK, C), lambda i: (i, 0)),
            pl.BlockSpec((_TBLK, 1), lambda i: (i, 0)),
            pl.BlockSpec((1, 1), lambda i: (0, 0)),
        ],
        out_shape=[
            jax.ShapeDtypeStruct((N, _NUM_EMB), jnp.float32),
            jax.ShapeDtypeStruct((N, C), jnp.float32),
            jax.ShapeDtypeStruct((N, 1), jnp.int32),
            jax.ShapeDtypeStruct((1, 1), jnp.float32),
        ],
        scratch_shapes=[
            pltpu.SMEM((1,), jnp.float32),
        ],
    )(flat, W, w2, wsq, iota)

    ones = jnp.ones((_PER_WORKER,), jnp.float32)
    zeros = jnp.zeros((_NUM_EMB,), jnp.float32)
    counts2 = _hist_sc(idx.reshape(N), ones, zeros)
    avg = (counts2[0] + counts2[1]) / N
    ppl = jnp.exp(-jnp.sum(avg * jnp.log(avg + 1e-10)))

    quantized_st = jnp.transpose(qst.reshape(B, H, Wd, C), (0, 3, 1, 2))
    return (loss[0, 0], quantized_st, ppl, enc)


# speedup vs baseline: 1.1049x; 1.1049x over previous
import jax
import jax.numpy as jnp
from jax import lax
from jax.experimental import pallas as pl
from jax.experimental.pallas import tpu as pltpu

_NUM_EMB = 1024
_EMB_DIM = 64
_COMMIT = 0.25


def _vq_body(x_ref, w_ref, w2_ref, wsq_ref, iota_r_ref, iota_c_ref, enc_ref,
             qst_ref, loss_ref, ppl_ref, sse_ref, cnt_ref):
    i = pl.program_id(0)
    nsteps = pl.num_programs(0)
    xb = x_ref[0]
    w = w_ref[...]
    xsq = jnp.sum(xb * xb, axis=0, keepdims=True)
    m2 = lax.dot_general(w2_ref[...], xb, (((1,), (0,)), ((), ())),
                         preferred_element_type=jnp.float32)
    dist = (xsq + wsq_ref[...]) - m2
    dmin = jnp.min(dist, axis=0, keepdims=True)
    iota_c = iota_c_ref[...]
    idx_t = jnp.min(jnp.where(dist == dmin, iota_c, float(_NUM_EMB)),
                    axis=0, keepdims=True)
    idx = lax.transpose(idx_t, (1, 0))
    enc = (iota_r_ref[...] == idx).astype(jnp.float32)
    enc_ref[...] = enc
    q = lax.dot_general(w, enc, (((0,), (1,)), ((), ())),
                        preferred_element_type=jnp.float32)
    d = q - xb
    qst_ref[0] = xb + d
    sse_part = jnp.sum(d * d)
    ones_row = jnp.full((1, enc.shape[0]), 1.0, jnp.float32)
    cnt_part = lax.dot_general(ones_row, enc, (((1,), (0,)), ((), ())),
                               preferred_element_type=jnp.float32)

    @pl.when(i == 0)
    def _init():
        sse_ref[0] = sse_part
        cnt_ref[...] = cnt_part

    @pl.when(i != 0)
    def _acc():
        sse_ref[0] += sse_part
        cnt_ref[...] += cnt_part

    @pl.when(i == nsteps - 1)
    def _fin():
        n_tok = nsteps * enc.shape[0]
        mean = sse_ref[0] / (n_tok * _EMB_DIM)
        loss_ref[...] = jnp.reshape(mean + _COMMIT * mean, (1, 1))
        avg = cnt_ref[...] / n_tok
        ent = jnp.sum(avg * jnp.log(avg + 1e-10), axis=1, keepdims=True)
        ppl_ref[...] = jnp.exp(-ent)


def kernel(inputs, W):
    B, C, H, Wd = inputs.shape
    HW = H * Wd
    N = B * HW
    xv = inputs.reshape(B, C, HW)
    wsq = jnp.sum(W ** 2, axis=1).reshape(_NUM_EMB, 1)
    w2 = W + W
    iota_r = lax.broadcasted_iota(jnp.float32, (1, _NUM_EMB), 1)
    iota_c = lax.broadcasted_iota(jnp.float32, (_NUM_EMB, 1), 0)

    enc, qst, loss, ppl = pl.pallas_call(
        _vq_body,
        grid=(B,),
        in_specs=[
            pl.BlockSpec((1, C, HW), lambda i: (i, 0, 0)),
            pl.BlockSpec((_NUM_EMB, C), lambda i: (0, 0)),
            pl.BlockSpec((_NUM_EMB, C), lambda i: (0, 0)),
            pl.BlockSpec((_NUM_EMB, 1), lambda i: (0, 0)),
            pl.BlockSpec((1, _NUM_EMB), lambda i: (0, 0)),
            pl.BlockSpec((_NUM_EMB, 1), lambda i: (0, 0)),
        ],
        out_specs=[
            pl.BlockSpec((HW, _NUM_EMB), lambda i: (i, 0)),
            pl.BlockSpec((1, C, HW), lambda i: (i, 0, 0)),
            pl.BlockSpec((1, 1), lambda i: (0, 0)),
            pl.BlockSpec((1, 1), lambda i: (0, 0)),
        ],
        out_shape=[
            jax.ShapeDtypeStruct((N, _NUM_EMB), jnp.float32),
            jax.ShapeDtypeStruct((B, C, HW), jnp.float32),
            jax.ShapeDtypeStruct((1, 1), jnp.float32),
            jax.ShapeDtypeStruct((1, 1), jnp.float32),
        ],
        scratch_shapes=[
            pltpu.SMEM((1,), jnp.float32),
            pltpu.VMEM((1, _NUM_EMB), jnp.float32),
        ],
    )(xv, W, w2, wsq, iota_r, iota_c)

    quantized_st = qst.reshape(B, C, H, Wd)
    return (loss[0, 0], quantized_st, ppl[0, 0], enc)
